# baseline (speedup 1.0000x reference)
import jax
import jax.numpy as jnp
from jax.experimental import pallas as pl

_K = 1024
_D = 256
_TILE = 512
_LOSS_W = 1.25


def _stage_kernel(r_ref, rsum_ref, cb_ref, cb2_ref, idx_ref, q_ref, sse_ref):
    i = pl.program_id(0)

    @pl.when(i == 0)
    def _init():
        sse_ref[...] = jnp.zeros((1, 1), jnp.float32)

    r = r_ref[...]
    cb = cb_ref[...]
    m = jax.lax.dot_general(r, cb, (((1,), (1,)), ((), ())),
                            preferred_element_type=jnp.float32)
    d2 = (rsum_ref[...] - 2.0 * m) + cb2_ref[...]
    lane = jax.lax.broadcasted_iota(jnp.int32, (_TILE, _K), 1)
    dmin = jnp.min(d2, axis=1, keepdims=True)
    idx = jnp.min(jnp.where(d2 == dmin, lane, _K), axis=1).astype(jnp.int32)
    onehot = (lane == idx[:, None]).astype(jnp.float32)
    q = jax.lax.dot_general(onehot, cb, (((1,), (0,)), ((), ())),
                            preferred_element_type=jnp.float32,
                            precision=jax.lax.Precision.HIGHEST)
    diff = q - r
    idx_ref[...] = idx[:, None]
    q_ref[...] = q
    sse_ref[...] = sse_ref[...] + jnp.reshape(jnp.sum(diff * diff), (1, 1))


def _stage(residual, rsum, cb, cb2):
    N, D = residual.shape
    return pl.pallas_call(
        _stage_kernel,
        grid=(N // _TILE,),
        in_specs=[
            pl.BlockSpec((_TILE, D), lambda i: (i, 0)),
            pl.BlockSpec((_TILE, 1), lambda i: (i, 0)),
            pl.BlockSpec((_K, D), lambda i: (0, 0)),
            pl.BlockSpec((1, _K), lambda i: (0, 0)),
        ],
        out_specs=[
            pl.BlockSpec((_TILE, 1), lambda i: (i, 0)),
            pl.BlockSpec((_TILE, D), lambda i: (i, 0)),
            pl.BlockSpec((1, 1), lambda i: (0, 0)),
        ],
        out_shape=[
            jax.ShapeDtypeStruct((N, 1), jnp.int32),
            jax.ShapeDtypeStruct((N, D), jnp.float32),
            jax.ShapeDtypeStruct((1, 1), jnp.float32),
        ],
    )(residual, rsum, cb, cb2)


def kernel(x, codebooks):
    B, T, D = x.shape
    nq = codebooks.shape[0]
    N = B * T
    flat = x.reshape(N, D)

    residual = flat
    quantized_sum = jnp.zeros_like(flat)
    indices_list = []
    sse_total = jnp.float32(0.0)
    for i in range(nq):
        cb = codebooks[i]
        rsum = jnp.sum(residual ** 2, axis=1, keepdims=True)
        cb2 = jnp.sum(cb ** 2, axis=1)[None, :]
        idx, q, sse = _stage(residual, rsum, cb, cb2)
        quantized_sum = quantized_sum + q
        residual = residual - q
        indices_list.append(idx[:, 0])
        sse_total = sse_total + sse[0, 0]

    quantized = (flat + (quantized_sum - flat)).reshape(B, T, D)
    indices = jnp.stack(indices_list, axis=-1).reshape(B, T, nq)
    vq_loss = sse_total * (_LOSS_W / (N * D))
    losses = jnp.full((nq,), vq_loss, dtype=jnp.float32)
    return (quantized, indices, losses)


# speedup vs baseline: 1.3042x; 1.3042x over previous
import functools

import jax
import jax.numpy as jnp
from jax.experimental import pallas as pl

_K = 1024
_D = 256
_TILE = 1024
_LOSS_W = 1.25


def _stage_kernel(r_ref, rsum_ref, cb_ref, limbs_ref, cb2_ref,
                  idx_ref, rnext_ref, sse_ref, *, last, flat_ref=None):
    i = pl.program_id(0)

    @pl.when(i == 0)
    def _init():
        sse_ref[...] = jnp.zeros((1, 1), jnp.float32)

    r = r_ref[...]
    m = jax.lax.dot_general(r, cb_ref[...], (((1,), (1,)), ((), ())),
                            preferred_element_type=jnp.float32)
    d2 = (rsum_ref[...] - 2.0 * m) + cb2_ref[...]
    lane = jax.lax.broadcasted_iota(jnp.int32, (_TILE, _K), 1)
    dmin = jnp.min(d2, axis=1, keepdims=True)
    idx = jnp.min(jnp.where(d2 == dmin, lane, _K), axis=1).astype(jnp.int32)
    lane3 = jax.lax.broadcasted_iota(jnp.int32, (_TILE, 3 * _K), 1)
    i3 = idx[:, None]
    threehot = ((lane3 == i3) | (lane3 == i3 + _K)
                | (lane3 == i3 + 2 * _K)).astype(jnp.float32)
    q = jax.lax.dot_general(threehot, limbs_ref[...], (((1,), (0,)), ((), ())),
                            preferred_element_type=jnp.float32)
    diff = q - r
    idx_ref[...] = idx[:, None]
    if last:
        rnext_ref[...] = flat_ref[...] - (r - q)
    else:
        rnext_ref[...] = r - q
    sse_ref[...] = sse_ref[...] + jnp.reshape(jnp.sum(diff * diff), (1, 1))


def _stage(residual, rsum, cb, limbs, cb2, flat=None):
    N, D = residual.shape
    last = flat is not None
    block_in = [
        pl.BlockSpec((_TILE, D), lambda i: (i, 0)),
        pl.BlockSpec((_TILE, 1), lambda i: (i, 0)),
        pl.BlockSpec((_K, D), lambda i: (0, 0)),
        pl.BlockSpec((3 * _K, D), lambda i: (0, 0)),
        pl.BlockSpec((1, _K), lambda i: (0, 0)),
    ]
    args = [residual, rsum, cb, limbs, cb2]
    if last:
        block_in.append(pl.BlockSpec((_TILE, D), lambda i: (i, 0)))
        args.append(flat)
        body = functools.partial(_reordered_last_kernel, last=True)
    else:
        body = functools.partial(_stage_kernel, last=False)
    return pl.pallas_call(
        body,
        grid=(N // _TILE,),
        in_specs=block_in,
        out_specs=[
            pl.BlockSpec((_TILE, 1), lambda i: (i, 0)),
            pl.BlockSpec((_TILE, D), lambda i: (i, 0)),
            pl.BlockSpec((1, 1), lambda i: (0, 0)),
        ],
        out_shape=[
            jax.ShapeDtypeStruct((N, 1), jnp.int32),
            jax.ShapeDtypeStruct((N, D), jnp.float32),
            jax.ShapeDtypeStruct((1, 1), jnp.float32),
        ],
    )(*args)


def _reordered_last_kernel(r_ref, rsum_ref, cb_ref, limbs_ref,
                           cb2_ref, flat_ref, idx_ref, rnext_ref, sse_ref, *,
                           last):
    _stage_kernel(r_ref, rsum_ref, cb_ref, limbs_ref, cb2_ref,
                  idx_ref, rnext_ref, sse_ref, last=last, flat_ref=flat_ref)


def _trunc_bf16(v):
    return jax.lax.bitcast_convert_type(
        jax.lax.bitcast_convert_type(v, jnp.uint32) & jnp.uint32(0xFFFF0000),
        jnp.float32)


def _split_limbs(cb):
    hi = _trunc_bf16(cb)
    rem = cb - hi
    mid = _trunc_bf16(rem)
    lo = rem - mid
    return jnp.concatenate([hi, mid, lo], axis=0)


def kernel(x, codebooks):
    B, T, D = x.shape
    nq = codebooks.shape[0]
    N = B * T
    flat = x.reshape(N, D)

    residual = flat
    indices_list = []
    sse_total = jnp.float32(0.0)
    for i in range(nq):
        cb = codebooks[i]
        limbs = _split_limbs(cb)
        rsum = jnp.sum(residual ** 2, axis=1, keepdims=True)
        cb2 = jnp.sum(cb ** 2, axis=1)[None, :]
        idx, nxt, sse = _stage(residual, rsum, cb, limbs, cb2,
                               flat=flat if i == nq - 1 else None)
        residual = nxt
        indices_list.append(idx[:, 0])
        sse_total = sse_total + sse[0, 0]

    quantized = residual.reshape(B, T, D)
    indices = jnp.stack(indices_list, axis=-1).reshape(B, T, nq)
    vq_loss = sse_total * (_LOSS_W / (N * D))
    losses = jnp.full((nq,), vq_loss, dtype=jnp.float32)
    return (quantized, indices, losses)


# speedup vs baseline: 1.4786x; 1.1337x over previous
import functools

import jax
import jax.numpy as jnp
from jax.experimental import pallas as pl

_K = 1024
_D = 256
_TILE = 1024
_LOSS_W = 1.25


def _stage_kernel(r_ref, rsum_ref, cb_ref, limbs_ref, cb2_ref,
                  idx_ref, rnext_ref, sse_ref, *, last, flat_ref=None):
    i = pl.program_id(0)

    @pl.when(i == 0)
    def _init():
        sse_ref[...] = jnp.zeros((1, 1), jnp.float32)

    r = r_ref[...]
    m = jax.lax.dot_general(r, cb_ref[...], (((1,), (1,)), ((), ())),
                            preferred_element_type=jnp.float32)
    d2 = (rsum_ref[...] - 2.0 * m) + cb2_ref[...]
    lane = jax.lax.broadcasted_iota(jnp.int32, (_TILE, _K), 1)
    dmin = jnp.min(d2, axis=1, keepdims=True)
    idx = jnp.min(jnp.where(d2 == dmin, lane, _K), axis=1).astype(jnp.int32)
    onehot = (lane == idx[:, None]).astype(jnp.float32)
    q3 = jax.lax.dot_general(onehot, limbs_ref[...], (((1,), (0,)), ((), ())),
                             preferred_element_type=jnp.float32)
    q = (q3[:, :_D] + q3[:, _D:2 * _D]) + q3[:, 2 * _D:]
    diff = q - r
    idx_ref[...] = idx[:, None]
    if last:
        rnext_ref[...] = flat_ref[...] - (r - q)
    else:
        rnext_ref[...] = r - q
    sse_ref[...] = sse_ref[...] + jnp.reshape(jnp.sum(diff * diff), (1, 1))


def _stage(residual, rsum, cb, limbs, cb2, flat=None):
    N, D = residual.shape
    last = flat is not None
    block_in = [
        pl.BlockSpec((_TILE, D), lambda i: (i, 0)),
        pl.BlockSpec((_TILE, 1), lambda i: (i, 0)),
        pl.BlockSpec((_K, D), lambda i: (0, 0)),
        pl.BlockSpec((_K, 3 * D), lambda i: (0, 0)),
        pl.BlockSpec((1, _K), lambda i: (0, 0)),
    ]
    args = [residual, rsum, cb, limbs, cb2]
    if last:
        block_in.append(pl.BlockSpec((_TILE, D), lambda i: (i, 0)))
        args.append(flat)
        body = functools.partial(_reordered_last_kernel, last=True)
    else:
        body = functools.partial(_stage_kernel, last=False)
    return pl.pallas_call(
        body,
        grid=(N // _TILE,),
        in_specs=block_in,
        out_specs=[
            pl.BlockSpec((_TILE, 1), lambda i: (i, 0)),
            pl.BlockSpec((_TILE, D), lambda i: (i, 0)),
            pl.BlockSpec((1, 1), lambda i: (0, 0)),
        ],
        out_shape=[
            jax.ShapeDtypeStruct((N, 1), jnp.int32),
            jax.ShapeDtypeStruct((N, D), jnp.float32),
            jax.ShapeDtypeStruct((1, 1), jnp.float32),
        ],
    )(*args)


def _reordered_last_kernel(r_ref, rsum_ref, cb_ref, limbs_ref,
                           cb2_ref, flat_ref, idx_ref, rnext_ref, sse_ref, *,
                           last):
    _stage_kernel(r_ref, rsum_ref, cb_ref, limbs_ref, cb2_ref,
                  idx_ref, rnext_ref, sse_ref, last=last, flat_ref=flat_ref)


def _trunc_bf16(v):
    return jax.lax.bitcast_convert_type(
        jax.lax.bitcast_convert_type(v, jnp.uint32) & jnp.uint32(0xFFFF0000),
        jnp.float32)


def _split_limbs(cb):
    hi = _trunc_bf16(cb)
    rem = cb - hi
    mid = _trunc_bf16(rem)
    lo = rem - mid
    return jnp.concatenate([hi, mid, lo], axis=1)


def kernel(x, codebooks):
    B, T, D = x.shape
    nq = codebooks.shape[0]
    N = B * T
    flat = x.reshape(N, D)

    residual = flat
    indices_list = []
    sse_total = jnp.float32(0.0)
    for i in range(nq):
        cb = codebooks[i]
        limbs = _split_limbs(cb)
        rsum = jnp.sum(residual ** 2, axis=1, keepdims=True)
        cb2 = jnp.sum(cb ** 2, axis=1)[None, :]
        idx, nxt, sse = _stage(residual, rsum, cb, limbs, cb2,
                               flat=flat if i == nq - 1 else None)
        residual = nxt
        indices_list.append(idx[:, 0])
        sse_total = sse_total + sse[0, 0]

    quantized = residual.reshape(B, T, D)
    indices = jnp.stack(indices_list, axis=-1).reshape(B, T, nq)
    vq_loss = sse_total * (_LOSS_W / (N * D))
    losses = jnp.full((nq,), vq_loss, dtype=jnp.float32)
    return (quantized, indices, losses)
